# Initial kernel scaffold; baseline (speedup 1.0000x reference)
import jax
import jax.numpy as jnp
from jax.experimental import pallas as pl


def kernel(frame_index, table):
    raise NotImplementedError("write your pallas kernel here")


# speedup vs baseline: 4.6240x; 4.6240x over previous
import functools

import jax
import jax.numpy as jnp
from jax import lax
from jax.experimental import pallas as pl
from jax.experimental.pallas import tpu as pltpu
from jax.experimental.pallas import tpu_sc as plsc

B = 16384
P = 10
NFRAMES = 20
D = 32
OFFSET = 500
NC = 2
NS = 16
NW = NC * NS
BPW = B // NW
CHUNK = 128
NCHUNK = BPW // CHUNK

_mesh = plsc.VectorSubcoreMesh(core_axis_name="c", subcore_axis_name="s")


@functools.partial(
    pl.kernel,
    out_type=jax.ShapeDtypeStruct((P * B, D), jnp.float32),
    mesh=_mesh,
    compiler_params=pltpu.CompilerParams(
        needs_layout_passes=False, use_tc_tiling_on_sc=False),
    scratch_types=[
        pltpu.VMEM((BPW * NFRAMES,), jnp.int32),
        pltpu.VMEM((P * BPW,), jnp.int32),
        pltpu.VMEM((BPW, D), jnp.float32),
        pltpu.SemaphoreType.DMA,
    ],
)
def _sc_lookup(fi_hbm, table_hbm, out_hbm, fi_v, idx_v, rows_v, sem):
    wid = lax.axis_index("s") * NC + lax.axis_index("c")
    base_b = wid * BPW
    pltpu.sync_copy(fi_hbm.at[pl.ds(base_b * NFRAMES, BPW * NFRAMES)], fi_v)

    lanes = lax.iota(jnp.int32, 16)

    def compute_idx(k, _):
        p = k // (BPW // 16)
        j = k - p * (BPW // 16)
        addr = (lanes + j * 16) * NFRAMES + p
        nxt = plsc.load_gather(fi_v, [addr + 10])
        prv = plsc.load_gather(fi_v, [addr])
        idx_v[pl.ds(k * 16, 16)] = nxt - prv + OFFSET
        return 0

    lax.fori_loop(0, P * (BPW // 16), compute_idx, 0)

    def per_p(p, _):
        copies = []
        for c in range(NCHUNK):
            copies.append(pltpu.async_copy(
                table_hbm.at[idx_v.at[pl.ds(p * BPW + c * CHUNK, CHUNK)]],
                rows_v.at[pl.ds(c * CHUNK, CHUNK), :],
                sem))
        for cp in copies:
            cp.wait()
        pltpu.sync_copy(rows_v, out_hbm.at[pl.ds(p * B + base_b, BPW), :])
        return 0

    lax.fori_loop(0, P, per_p, 0)


def kernel(frame_index, table):
    fi = frame_index.reshape(B * NFRAMES)
    out = _sc_lookup(fi, table)
    return out.reshape(P * B, 1, 1, D)


# speedup vs baseline: 4.6641x; 1.0087x over previous
import functools

import jax
import jax.numpy as jnp
from jax import lax
from jax.experimental import pallas as pl
from jax.experimental.pallas import tpu as pltpu
from jax.experimental.pallas import tpu_sc as plsc

B = 16384
P = 10
NFRAMES = 20
D = 32
OFFSET = 500
NC = 2
NS = 16
NW = NC * NS
BPW = B // NW
CHUNK = 512
NCHUNK = BPW // CHUNK

_mesh = plsc.VectorSubcoreMesh(core_axis_name="c", subcore_axis_name="s")


@functools.partial(
    pl.kernel,
    out_type=jax.ShapeDtypeStruct((P * B, D), jnp.float32),
    mesh=_mesh,
    compiler_params=pltpu.CompilerParams(
        needs_layout_passes=False, use_tc_tiling_on_sc=False),
    scratch_types=[
        pltpu.VMEM((BPW * NFRAMES,), jnp.int32),
        pltpu.VMEM((P * BPW,), jnp.int32),
        pltpu.VMEM((BPW, D), jnp.float32),
        pltpu.VMEM((BPW, D), jnp.float32),
        pltpu.SemaphoreType.DMA,
        pltpu.SemaphoreType.DMA,
        pltpu.SemaphoreType.DMA,
        pltpu.SemaphoreType.DMA,
    ],
)
def _sc_lookup(fi_hbm, table_hbm, out_hbm, fi_v, idx_v,
               rows_v0, rows_v1, gsem0, gsem1, ssem0, ssem1):
    wid = lax.axis_index("s") * NC + lax.axis_index("c")
    base_b = wid * BPW
    pltpu.sync_copy(fi_hbm.at[pl.ds(base_b * NFRAMES, BPW * NFRAMES)], fi_v)

    lanes = lax.iota(jnp.int32, 16)

    def compute_idx(k, _):
        p = k // (BPW // 16)
        j = k - p * (BPW // 16)
        addr = (lanes + j * 16) * NFRAMES + p
        nxt = plsc.load_gather(fi_v, [addr + 10])
        prv = plsc.load_gather(fi_v, [addr])
        idx_v[pl.ds(k * 16, 16)] = nxt - prv + OFFSET
        return 0

    lax.fori_loop(0, P * (BPW // 16), compute_idx, 0)

    rows = (rows_v0, rows_v1)
    gsem = (gsem0, gsem1)
    ssem = (ssem0, ssem1)

    def fire_gather(p, b):
        return [
            pltpu.async_copy(
                table_hbm.at[idx_v.at[pl.ds(p * BPW + c * CHUNK, CHUNK)]],
                rows[b].at[pl.ds(c * CHUNK, CHUNK), :],
                gsem[b])
            for c in range(NCHUNK)
        ]

    def fire_scatter(p, b):
        return pltpu.async_copy(
            rows[b], out_hbm.at[pl.ds(p * B + base_b, BPW), :], ssem[b])

    gath = [None, None]
    scat = [None, None]
    for p in range(P):
        b = p & 1
        if scat[b] is not None:
            scat[b].wait()
            scat[b] = None
        gath[b] = fire_gather(p, b)
        if p >= 1:
            bb = (p - 1) & 1
            for cp in gath[bb]:
                cp.wait()
            scat[bb] = fire_scatter(p - 1, bb)
    bl = (P - 1) & 1
    for cp in gath[bl]:
        cp.wait()
    scat[bl] = fire_scatter(P - 1, bl)
    scat[0].wait()
    scat[1].wait()


def kernel(frame_index, table):
    fi = frame_index.reshape(B * NFRAMES)
    out = _sc_lookup(fi, table)
    return out.reshape(P * B, 1, 1, D)


# speedup vs baseline: 6.2818x; 1.3468x over previous
import functools

import jax
import jax.numpy as jnp
from jax import lax
from jax.experimental import pallas as pl
from jax.experimental.pallas import tpu as pltpu
from jax.experimental.pallas import tpu_sc as plsc

B = 16384
P = 10
NFRAMES = 20
D = 32
NROWS = 1000
OFFSET = 500
NC = 2
NS = 16
NW = NC * NS
BPW = B // NW
BLK = 256
NBLK = P * BPW // BLK

_mesh = plsc.VectorSubcoreMesh(core_axis_name="c", subcore_axis_name="s")


@functools.partial(
    pl.kernel,
    out_type=jax.ShapeDtypeStruct((P * B, D), jnp.float32),
    mesh=_mesh,
    compiler_params=pltpu.CompilerParams(
        needs_layout_passes=False, use_tc_tiling_on_sc=True),
    scratch_types=[
        pltpu.VMEM((BPW * NFRAMES,), jnp.int32),
        pltpu.VMEM((NROWS * D,), jnp.float32),
        pltpu.VMEM((P * BPW,), jnp.int32),
        pltpu.VMEM((BLK, D), jnp.float32),
        pltpu.VMEM((BLK, D), jnp.float32),
        pltpu.SemaphoreType.DMA,
        pltpu.SemaphoreType.DMA,
        pltpu.SemaphoreType.DMA,
    ],
)
def _sc_lookup(fi_hbm, table_hbm, out_hbm, fi_v, tab_v, idx_v,
               rows_v0, rows_v1, tsem, ssem0, ssem1):
    wid = lax.axis_index("s") * NC + lax.axis_index("c")
    base_b = wid * BPW

    tcopy = pltpu.async_copy(table_hbm, tab_v, tsem)
    pltpu.sync_copy(fi_hbm.at[pl.ds(base_b * NFRAMES, BPW * NFRAMES)], fi_v)

    lanes = lax.iota(jnp.int32, 16)

    def compute_idx(k, _):
        p = k // (BPW // 16)
        j = k - p * (BPW // 16)
        addr = (lanes + j * 16) * NFRAMES + p
        nxt = plsc.load_gather(fi_v, [addr + 10])
        prv = plsc.load_gather(fi_v, [addr])
        idx_v[pl.ds(k * 16, 16)] = (nxt - prv + OFFSET) * D
        return 0

    lax.fori_loop(0, P * (BPW // 16), compute_idx, 0)
    tcopy.wait()

    rows = (rows_v0, rows_v1)
    ssem = (ssem0, ssem1)

    def lookup_block(t, buf):
        def one_row(j, _):
            base = plsc.load_gather(
                idx_v, [jnp.full((16,), t * BLK, jnp.int32) + j])
            lo = plsc.load_gather(tab_v, [base + lanes])
            hi = plsc.load_gather(tab_v, [base + (lanes + 16)])
            buf[j, pl.ds(0, 16)] = lo
            buf[j, pl.ds(16, 16)] = hi
            return 0

        lax.fori_loop(0, BLK, one_row, 0)

    scat = [None, None]
    for t in range(NBLK):
        bsel = t & 1
        if scat[bsel] is not None:
            scat[bsel].wait()
            scat[bsel] = None
        lookup_block(t, rows[bsel])
        p, h = divmod(t, BPW // BLK)
        dst_row = p * B + base_b + h * BLK
        scat[bsel] = pltpu.async_copy(
            rows[bsel], out_hbm.at[pl.ds(dst_row, BLK), :], ssem[bsel])
    scat[0].wait()
    scat[1].wait()


def kernel(frame_index, table):
    fi = frame_index.reshape(B * NFRAMES)
    out = _sc_lookup(fi, table.reshape(NROWS * D))
    return out.reshape(P * B, 1, 1, D)


# speedup vs baseline: 10.5038x; 1.6721x over previous
import functools

import jax
import jax.numpy as jnp
from jax import lax
from jax.experimental import pallas as pl
from jax.experimental.pallas import tpu as pltpu
from jax.experimental.pallas import tpu_sc as plsc

B = 16384
P = 10
NFRAMES = 20
D = 32
NROWS = 1000
OFFSET = 500
NC = 2
NS = 16
NW = NC * NS
BPW = B // NW

_mesh = plsc.VectorSubcoreMesh(core_axis_name="c", subcore_axis_name="s")


@functools.partial(
    pl.kernel,
    out_type=jax.ShapeDtypeStruct((D, P * B), jnp.float32),
    mesh=_mesh,
    compiler_params=pltpu.CompilerParams(
        needs_layout_passes=False, use_tc_tiling_on_sc=True),
    scratch_types=[
        pltpu.VMEM((NFRAMES * BPW,), jnp.int32),
        pltpu.VMEM((D * NROWS,), jnp.float32),
        pltpu.VMEM((P * BPW,), jnp.int32),
        pltpu.VMEM((D // 8, 8, BPW), jnp.float32),
        pltpu.VMEM((D // 8, 8, BPW), jnp.float32),
        pltpu.SemaphoreType.DMA,
        pltpu.SemaphoreType.DMA,
        pltpu.SemaphoreType.DMA,
    ],
)
def _sc_lookup(fi_hbm, table_hbm, out_hbm, fi_v, tab_v, idx_v,
               buf0, buf1, fsem, ssem0, ssem1):
    wid = lax.axis_index("s") * NC + lax.axis_index("c")
    base_b = wid * BPW

    tcopy = pltpu.async_copy(table_hbm, tab_v, fsem)
    fcopies = [
        pltpu.async_copy(fi_hbm.at[pl.ds(f * B + base_b, BPW)],
                         fi_v.at[pl.ds(f * BPW, BPW)], fsem)
        for f in range(NFRAMES)
    ]
    tcopy.wait()
    for cp in fcopies:
        cp.wait()

    def compute_idx(k, _):
        p = k // (BPW // 16)
        j = k - p * (BPW // 16)
        nxt = fi_v[pl.ds((p + 10) * BPW + j * 16, 16)]
        prv = fi_v[pl.ds(p * BPW + j * 16, 16)]
        idx_v[pl.ds(k * 16, 16)] = nxt - prv + OFFSET
        return 0

    lax.fori_loop(0, P * (BPW // 16), compute_idx, 0)

    bufs = (buf0, buf1)
    ssem = (ssem0, ssem1)

    def lookup_block(p, buf):
        def group(j0, _):
            idx = idx_v[pl.ds(p * BPW + j0 * 16, 16)]
            for c in range(D):
                vals = plsc.load_gather(tab_v, [idx + c * NROWS])
                buf[c // 8, c % 8, pl.ds(j0 * 16, 16)] = vals
            return 0

        lax.fori_loop(0, BPW // 16, group, 0)

    scat = [None, None]
    for p in range(P):
        bsel = p & 1
        if scat[bsel] is not None:
            for cp in scat[bsel]:
                cp.wait()
            scat[bsel] = None
        lookup_block(p, bufs[bsel])
        r0 = p * B + base_b
        scat[bsel] = [
            pltpu.async_copy(
                bufs[bsel].at[cg],
                out_hbm.at[pl.ds(cg * 8, 8), pl.ds(r0, BPW)],
                ssem[bsel])
            for cg in range(D // 8)
        ]
    for s in scat:
        for cp in s:
            cp.wait()


def kernel(frame_index, table):
    fi_t = jnp.transpose(frame_index, (3, 1, 2, 0)).reshape(NFRAMES * B)
    tab_t = jnp.transpose(table).reshape(D * NROWS)
    out_t = _sc_lookup(fi_t, tab_t)
    return jnp.transpose(out_t).reshape(P * B, 1, 1, D)


# speedup vs baseline: 21.1930x; 2.0176x over previous
import functools

import jax
import jax.numpy as jnp
from jax import lax
from jax.experimental import pallas as pl
from jax.experimental.pallas import tpu as pltpu
from jax.experimental.pallas import tpu_sc as plsc

B = 16384
P = 10
NFRAMES = 20
D = 32
NROWS = 1000
OFFSET = 500
NC = 2
NS = 16
NW = NC * NS
BPW = B // NW

_mesh = plsc.VectorSubcoreMesh(core_axis_name="c", subcore_axis_name="s")


@functools.partial(
    pl.kernel,
    out_type=jax.ShapeDtypeStruct((D, P * B), jnp.float32),
    mesh=_mesh,
    compiler_params=pltpu.CompilerParams(
        needs_layout_passes=False, use_tc_tiling_on_sc=True),
    scratch_types=[
        pltpu.VMEM((NFRAMES * BPW,), jnp.int32),
        pltpu.VMEM((D * NROWS,), jnp.float32),
        pltpu.VMEM((P * BPW,), jnp.int32),
        pltpu.VMEM((D // 8, 8, BPW), jnp.float32),
        pltpu.VMEM((D // 8, 8, BPW), jnp.float32),
        pltpu.SemaphoreType.DMA,
        pltpu.SemaphoreType.DMA,
        pltpu.SemaphoreType.DMA,
    ],
)
def _sc_lookup(fi_hbm, table_hbm, out_hbm, fi_v, tab_v, idx_v,
               buf0, buf1, fsem, ssem0, ssem1):
    wid = lax.axis_index("s") * NC + lax.axis_index("c")
    base_b = wid * BPW

    tcopy = pltpu.async_copy(table_hbm, tab_v, fsem)
    fcopies = [
        pltpu.async_copy(fi_hbm.at[pl.ds(f * B + base_b, BPW)],
                         fi_v.at[pl.ds(f * BPW, BPW)], fsem)
        for f in range(NFRAMES)
    ]
    tcopy.wait()
    for cp in fcopies:
        cp.wait()

    def compute_idx(k, _):
        p = k // (BPW // 16)
        j = k - p * (BPW // 16)
        nxt = fi_v[pl.ds((p + 10) * BPW + j * 16, 16)]
        prv = fi_v[pl.ds(p * BPW + j * 16, 16)]
        idx_v[pl.ds(k * 16, 16)] = nxt - prv + OFFSET
        return 0

    lax.fori_loop(0, P * (BPW // 16), compute_idx, 0)

    bufs = (buf0, buf1)
    ssem = (ssem0, ssem1)

    def lookup_block(p, buf):
        @plsc.parallel_loop(0, BPW // 16)
        def group(j0):
            idx = idx_v[pl.ds(p * BPW + j0 * 16, 16)]
            addrs = [idx + c * NROWS for c in range(D)]
            vals = [plsc.load_gather(tab_v, [a]) for a in addrs]
            for c in range(D):
                buf[c // 8, c % 8, pl.ds(j0 * 16, 16)] = vals[c]

    scat = [None, None]
    for p in range(P):
        bsel = p & 1
        if scat[bsel] is not None:
            for cp in scat[bsel]:
                cp.wait()
            scat[bsel] = None
        lookup_block(p, bufs[bsel])
        r0 = p * B + base_b
        scat[bsel] = [
            pltpu.async_copy(
                bufs[bsel].at[cg],
                out_hbm.at[pl.ds(cg * 8, 8), pl.ds(r0, BPW)],
                ssem[bsel])
            for cg in range(D // 8)
        ]
    for s in scat:
        for cp in s:
            cp.wait()


def kernel(frame_index, table):
    fi_t = jnp.transpose(frame_index, (3, 1, 2, 0)).reshape(NFRAMES * B)
    tab_t = jnp.transpose(table).reshape(D * NROWS)
    out_t = _sc_lookup(fi_t, tab_t)
    return jnp.transpose(out_t).reshape(P * B, 1, 1, D)
